# baseline (speedup 1.0000x reference)
import functools

import jax
import jax.numpy as jnp
from jax import lax
from jax.experimental import pallas as pl
from jax.experimental.pallas import tpu as pltpu
from jax.experimental.pallas import tpu_sc as plsc

_B = 8
_C = 128
_H = 256
_W = 256
_T = 16384
_HW = _H * _W
_TOT = _B * _T

_TBLK = 1024


def _tc_transpose(feat):
    eye = jnp.eye(_C, dtype=jnp.float32)

    def body(in_ref, eye_ref, out_ref):
        out_ref[0] = lax.dot_general(
            in_ref[0],
            eye_ref[...],
            (((0,), (0,)), ((), ())),
            preferred_element_type=jnp.float32,
        )

    return pl.pallas_call(
        body,
        grid=(_B, _HW // _TBLK),
        in_specs=[
            pl.BlockSpec((1, _C, _TBLK), lambda b, j: (b, 0, j)),
            pl.BlockSpec((_C, _C), lambda b, j: (0, 0)),
        ],
        out_specs=pl.BlockSpec((1, _TBLK, _C), lambda b, j: (b, j, 0)),
        out_shape=jax.ShapeDtypeStruct((_B, _HW, _C), jnp.float32),
    )(feat, eye)


def _sc_gather(table, tk_flat):
    info = plsc.get_sparse_core_info()
    nw = info.num_cores * info.num_subcores
    per_w = _TOT // nw
    chunk = 512

    mesh = plsc.VectorSubcoreMesh(core_axis_name="c", subcore_axis_name="s")

    @functools.partial(
        pl.kernel,
        mesh=mesh,
        out_type=jax.ShapeDtypeStruct((_TOT, _C), jnp.float32),
        scratch_types=[
            pltpu.VMEM((2 * per_w,), jnp.int32),
            pltpu.VMEM((per_w,), jnp.int32),
            pltpu.VMEM((chunk, _C), jnp.float32),
            pltpu.SemaphoreType.DMA,
        ],
        compiler_params=pltpu.CompilerParams(needs_layout_passes=False),
    )
    def k(table_hbm, tk_hbm, out_hbm, tk_v, idx_v, rows_v, sem):
        wid = lax.axis_index("s") * info.num_cores + lax.axis_index("c")
        g0 = wid * per_w
        batch = g0 // _T
        row_base = batch * _HW

        pltpu.sync_copy(tk_hbm.at[pl.ds(2 * g0, 2 * per_w)], tk_v)

        lanes = lax.iota(jnp.int32, 16)

        def compute_idx(i, _):
            base = i * 32
            xv = plsc.load_gather(tk_v, [base + lanes * 2])
            yv = plsc.load_gather(tk_v, [base + lanes * 2 + 1])
            idx_v[pl.ds(i * 16, 16)] = row_base + yv * _W + xv
            return 0

        lax.fori_loop(0, per_w // 16, compute_idx, 0)

        def gather_chunk(j, _):
            t0 = j * chunk
            pltpu.async_copy(
                table_hbm.at[idx_v.at[pl.ds(t0, chunk)]], rows_v, sem
            ).wait()
            pltpu.sync_copy(rows_v, out_hbm.at[pl.ds(g0 + t0, chunk)])
            return 0

        lax.fori_loop(0, per_w // chunk, gather_chunk, 0)

    return k(table, tk_flat)


def kernel(tk_codes, feat_grid):
    tk_flat = tk_codes.astype(jnp.int32).reshape(-1)
    table = _tc_transpose(feat_grid.reshape(_B, _C, _HW)).reshape(_B * _HW, _C)
    out = _sc_gather(table, tk_flat)
    return out.reshape(_B, _T, _C)


# speedup vs baseline: 1.9309x; 1.9309x over previous
import functools

import jax
import jax.numpy as jnp
from jax import lax
from jax.experimental import pallas as pl
from jax.experimental.pallas import tpu as pltpu
from jax.experimental.pallas import tpu_sc as plsc

_B = 8
_C = 128
_H = 256
_W = 256
_T = 16384
_HW = _H * _W
_TOT = _B * _T

_TBLK = 1024


def _tc_transpose(feat):
    eye = jnp.eye(_C, dtype=jnp.float32)

    def body(in_ref, eye_ref, out_ref):
        out_ref[0] = lax.dot_general(
            in_ref[0],
            eye_ref[...],
            (((0,), (0,)), ((), ())),
            preferred_element_type=jnp.float32,
        )

    return pl.pallas_call(
        body,
        grid=(_B, _HW // _TBLK),
        in_specs=[
            pl.BlockSpec((1, _C, _TBLK), lambda b, j: (b, 0, j)),
            pl.BlockSpec((_C, _C), lambda b, j: (0, 0)),
        ],
        out_specs=pl.BlockSpec((1, _TBLK, _C), lambda b, j: (b, j, 0)),
        out_shape=jax.ShapeDtypeStruct((_B, _HW, _C), jnp.float32),
    )(feat, eye)


def _sc_gather(table, tk_flat):
    info = plsc.get_sparse_core_info()
    nw = info.num_cores * info.num_subcores
    per_w = _TOT // nw
    chunk = 256
    nchunks = per_w // chunk

    mesh = plsc.VectorSubcoreMesh(core_axis_name="c", subcore_axis_name="s")

    @functools.partial(
        pl.kernel,
        mesh=mesh,
        out_type=jax.ShapeDtypeStruct((_TOT, _C), jnp.float32),
        scratch_types=[
            pltpu.VMEM((2 * per_w,), jnp.int32),
            pltpu.VMEM((per_w,), jnp.int32),
            pltpu.VMEM((chunk, _C), jnp.float32),
            pltpu.VMEM((chunk, _C), jnp.float32),
            pltpu.SemaphoreType.DMA,
            pltpu.SemaphoreType.DMA,
            pltpu.SemaphoreType.DMA,
            pltpu.SemaphoreType.DMA,
        ],
        compiler_params=pltpu.CompilerParams(needs_layout_passes=False),
    )
    def k(table_hbm, tk_hbm, out_hbm, tk_v, idx_v, rows0, rows1, gs0, gs1, ws0, ws1):
        wid = lax.axis_index("s") * info.num_cores + lax.axis_index("c")
        g0 = wid * per_w
        batch = g0 // _T
        row_base = batch * _HW

        pltpu.sync_copy(tk_hbm.at[pl.ds(2 * g0, 2 * per_w)], tk_v)

        lanes = lax.iota(jnp.int32, 16)

        def compute_idx(i, _):
            base = i * 32
            xv = plsc.load_gather(tk_v, [base + lanes * 2])
            yv = plsc.load_gather(tk_v, [base + lanes * 2 + 1])
            idx_v[pl.ds(i * 16, 16)] = row_base + yv * _W + xv
            return 0

        lax.fori_loop(0, per_w // 16, compute_idx, 0)

        rows = (rows0, rows1)
        gsem = (gs0, gs1)
        wsem = (ws0, ws1)

        def start_gather(j):
            return pltpu.async_copy(
                table_hbm.at[idx_v.at[pl.ds(j * chunk, chunk)]],
                rows[j % 2],
                gsem[j % 2],
            )

        def start_write(j):
            return pltpu.async_copy(
                rows[j % 2], out_hbm.at[pl.ds(g0 + j * chunk, chunk)], wsem[j % 2]
            )

        gd = [None] * nchunks
        wd = [None] * nchunks
        gd[0] = start_gather(0)
        for j in range(nchunks):
            if j + 1 < nchunks:
                if j - 1 >= 0:
                    wd[j - 1].wait()
                gd[j + 1] = start_gather(j + 1)
            gd[j].wait()
            wd[j] = start_write(j)
        wd[nchunks - 2].wait()
        wd[nchunks - 1].wait()

    return k(table, tk_flat)


def kernel(tk_codes, feat_grid):
    tk_flat = tk_codes.astype(jnp.int32).reshape(-1)
    table = feat_grid.reshape(_B * _HW, _C)
    out = _sc_gather(table, tk_flat)
    return out.reshape(_B, _T, _C)


# speedup vs baseline: 2.3890x; 1.2372x over previous
import functools

import jax
import jax.numpy as jnp
from jax import lax
from jax.experimental import pallas as pl
from jax.experimental.pallas import tpu as pltpu
from jax.experimental.pallas import tpu_sc as plsc

_B = 8
_C = 128
_H = 256
_W = 256
_T = 16384
_HW = _H * _W
_TOT = _B * _T

_YB = 32


def _tc_transpose(feat):
    eye = jnp.eye(_C, dtype=jnp.float32)
    blk = _YB * _W

    def body(in_ref, eye_ref, out_ref):
        a = in_ref[0].reshape(_C, blk)
        out_ref[...] = lax.dot_general(
            a,
            eye_ref[...],
            (((0,), (0,)), ((), ())),
            preferred_element_type=jnp.float32,
        )

    return pl.pallas_call(
        body,
        grid=(_B, _H // _YB),
        in_specs=[
            pl.BlockSpec((1, _C, _YB, _W), lambda b, j: (b, 0, j, 0)),
            pl.BlockSpec((_C, _C), lambda b, j: (0, 0)),
        ],
        out_specs=pl.BlockSpec((blk, _C), lambda b, j: (b * (_H // _YB) + j, 0)),
        out_shape=jax.ShapeDtypeStruct((_B * _HW, _C), jnp.float32),
    )(feat, eye)


def _sc_gather(table, tk_flat):
    info = plsc.get_sparse_core_info()
    nw = info.num_cores * info.num_subcores
    per_w = _TOT // nw
    chunk = 256
    nchunks = per_w // chunk

    mesh = plsc.VectorSubcoreMesh(core_axis_name="c", subcore_axis_name="s")

    @functools.partial(
        pl.kernel,
        mesh=mesh,
        out_type=jax.ShapeDtypeStruct((_B, _T, _C), jnp.float32),
        scratch_types=[
            pltpu.VMEM((2 * per_w,), jnp.int32),
            pltpu.VMEM((per_w,), jnp.int32),
            pltpu.VMEM((chunk, _C), jnp.float32),
            pltpu.VMEM((chunk, _C), jnp.float32),
            pltpu.SemaphoreType.DMA,
            pltpu.SemaphoreType.DMA,
            pltpu.SemaphoreType.DMA,
            pltpu.SemaphoreType.DMA,
        ],
        compiler_params=pltpu.CompilerParams(needs_layout_passes=False),
    )
    def k(table_hbm, tk_hbm, out_hbm, tk_v, idx_v, rows0, rows1, gs0, gs1, ws0, ws1):
        wid = lax.axis_index("s") * info.num_cores + lax.axis_index("c")
        g0 = wid * per_w
        batch = g0 // _T
        t0 = g0 % _T
        row_base = batch * _HW

        pltpu.sync_copy(tk_hbm.at[pl.ds(2 * g0, 2 * per_w)], tk_v)

        lanes = lax.iota(jnp.int32, 16)

        def compute_idx(i, _):
            base = i * 32
            xv = plsc.load_gather(tk_v, [base + lanes * 2])
            yv = plsc.load_gather(tk_v, [base + lanes * 2 + 1])
            idx_v[pl.ds(i * 16, 16)] = row_base + yv * _W + xv
            return 0

        lax.fori_loop(0, per_w // 16, compute_idx, 0)

        rows = (rows0, rows1)
        gsem = (gs0, gs1)
        wsem = (ws0, ws1)

        def start_gather(j):
            return pltpu.async_copy(
                table_hbm.at[idx_v.at[pl.ds(j * chunk, chunk)]],
                rows[j % 2],
                gsem[j % 2],
            )

        def start_write(j):
            return pltpu.async_copy(
                rows[j % 2],
                out_hbm.at[batch, pl.ds(t0 + j * chunk, chunk), :],
                wsem[j % 2],
            )

        gd = [None] * nchunks
        wd = [None] * nchunks
        gd[0] = start_gather(0)
        for j in range(nchunks):
            if j + 1 < nchunks:
                if j - 1 >= 0:
                    wd[j - 1].wait()
                gd[j + 1] = start_gather(j + 1)
            gd[j].wait()
            wd[j] = start_write(j)
        wd[nchunks - 2].wait()
        wd[nchunks - 1].wait()

    return k(table, tk_flat)


def kernel(tk_codes, feat_grid):
    tk_flat = tk_codes.astype(jnp.int32).reshape(-1)
    table = _tc_transpose(feat_grid)
    return _sc_gather(table, tk_flat)


# speedup vs baseline: 2.5016x; 1.0471x over previous
import functools

import jax
import jax.numpy as jnp
from jax import lax
from jax.experimental import pallas as pl
from jax.experimental.pallas import tpu as pltpu
from jax.experimental.pallas import tpu_sc as plsc

_B = 8
_C = 128
_H = 256
_W = 256
_T = 16384
_HW = _H * _W
_TOT = _B * _T

_YB = 32


def _tc_prep(feat, tk):
    eye = jnp.eye(_C, dtype=jnp.float32)
    blk = _YB * _W

    def body(in_ref, tk_ref, eye_ref, out_ref, idx_ref):
        a = in_ref[0].reshape(_C, blk)
        out_ref[...] = lax.dot_general(
            a,
            eye_ref[...],
            (((0,), (0,)), ((), ())),
            preferred_element_type=jnp.float32,
        )
        tkb = tk_ref[0]
        idx_ref[...] = pl.program_id(0) * _HW + tkb[:, 1] * _W + tkb[:, 0]

    return pl.pallas_call(
        body,
        grid=(_B, _H // _YB),
        in_specs=[
            pl.BlockSpec((1, _C, _YB, _W), lambda b, j: (b, 0, j, 0)),
            pl.BlockSpec((1, _T // (_H // _YB), 2), lambda b, j: (b, j, 0)),
            pl.BlockSpec((_C, _C), lambda b, j: (0, 0)),
        ],
        out_specs=[
            pl.BlockSpec((blk, _C), lambda b, j: (b * (_H // _YB) + j, 0)),
            pl.BlockSpec(
                (_T // (_H // _YB),), lambda b, j: (b * (_H // _YB) + j,)
            ),
        ],
        out_shape=[
            jax.ShapeDtypeStruct((_B * _HW, _C), jnp.float32),
            jax.ShapeDtypeStruct((_TOT,), jnp.int32),
        ],
    )(feat, tk, eye)


def _sc_gather(table, idx):
    info = plsc.get_sparse_core_info()
    nw = info.num_cores * info.num_subcores
    per_w = _TOT // nw
    chunk = 256
    nchunks = per_w // chunk

    mesh = plsc.VectorSubcoreMesh(core_axis_name="c", subcore_axis_name="s")

    @functools.partial(
        pl.kernel,
        mesh=mesh,
        out_type=jax.ShapeDtypeStruct((_B, _T, _C), jnp.float32),
        scratch_types=[
            pltpu.VMEM((per_w,), jnp.int32),
            pltpu.VMEM((chunk, _C), jnp.float32),
            pltpu.VMEM((chunk, _C), jnp.float32),
            pltpu.SemaphoreType.DMA,
            pltpu.SemaphoreType.DMA,
            pltpu.SemaphoreType.DMA,
            pltpu.SemaphoreType.DMA,
        ],
        compiler_params=pltpu.CompilerParams(needs_layout_passes=False),
    )
    def k(table_hbm, idx_hbm, out_hbm, idx_v, rows0, rows1, gs0, gs1, ws0, ws1):
        wid = lax.axis_index("s") * info.num_cores + lax.axis_index("c")
        g0 = wid * per_w
        batch = g0 // _T
        t0 = g0 % _T

        pltpu.sync_copy(idx_hbm.at[pl.ds(g0, per_w)], idx_v)

        rows = (rows0, rows1)
        gsem = (gs0, gs1)
        wsem = (ws0, ws1)

        def start_gather(j):
            return pltpu.async_copy(
                table_hbm.at[idx_v.at[pl.ds(j * chunk, chunk)]],
                rows[j % 2],
                gsem[j % 2],
            )

        def start_write(j):
            return pltpu.async_copy(
                rows[j % 2],
                out_hbm.at[batch, pl.ds(t0 + j * chunk, chunk), :],
                wsem[j % 2],
            )

        gd = [None] * nchunks
        wd = [None] * nchunks
        gd[0] = start_gather(0)
        for j in range(nchunks):
            if j + 1 < nchunks:
                if j - 1 >= 0:
                    wd[j - 1].wait()
                gd[j + 1] = start_gather(j + 1)
            gd[j].wait()
            wd[j] = start_write(j)
        wd[nchunks - 2].wait()
        wd[nchunks - 1].wait()

    return k(table, idx)


def kernel(tk_codes, feat_grid):
    tk = tk_codes.astype(jnp.int32)
    table, idx = _tc_prep(feat_grid, tk)
    return _sc_gather(table, idx)


# speedup vs baseline: 3.1140x; 1.2448x over previous
import functools

import jax
import jax.numpy as jnp
from jax import lax
from jax.experimental import pallas as pl
from jax.experimental.pallas import tpu as pltpu
from jax.experimental.pallas import tpu_sc as plsc

_B = 8
_C = 128
_H = 256
_W = 256
_T = 16384
_HW = _H * _W
_TOT = _B * _T

_YB = 32


def _tc_prep(feat, idx2d):
    eye = jnp.eye(_C, dtype=jnp.float32)
    blk = _YB * _W
    tblk = _T // (_H // _YB)

    def body(in_ref, idx_in_ref, eye_ref, out_ref, idx_ref):
        a = in_ref[0].reshape(_C, blk)
        out_ref[...] = lax.dot_general(
            a,
            eye_ref[...],
            (((0,), (0,)), ((), ())),
            preferred_element_type=jnp.float32,
        )
        idx_ref[...] = idx_in_ref[0, 0]

    return pl.pallas_call(
        body,
        grid=(_B, _H // _YB),
        in_specs=[
            pl.BlockSpec((1, _C, _YB, _W), lambda b, j: (b, 0, j, 0)),
            pl.BlockSpec((1, 1, tblk), lambda b, j: (b, 0, j)),
            pl.BlockSpec((_C, _C), lambda b, j: (0, 0)),
        ],
        out_specs=[
            pl.BlockSpec((blk, _C), lambda b, j: (b * (_H // _YB) + j, 0)),
            pl.BlockSpec((tblk,), lambda b, j: (b * (_H // _YB) + j,)),
        ],
        out_shape=[
            jax.ShapeDtypeStruct((_B * _HW, _C), jnp.float32),
            jax.ShapeDtypeStruct((_TOT,), jnp.int32),
        ],
    )(feat, idx2d, eye)


def _sc_gather(table, idx):
    info = plsc.get_sparse_core_info()
    nw = info.num_cores * info.num_subcores
    per_w = _TOT // nw
    chunk = 256
    nchunks = per_w // chunk

    mesh = plsc.VectorSubcoreMesh(core_axis_name="c", subcore_axis_name="s")

    @functools.partial(
        pl.kernel,
        mesh=mesh,
        out_type=jax.ShapeDtypeStruct((_B, _T, _C), jnp.float32),
        scratch_types=[
            pltpu.VMEM((per_w,), jnp.int32),
            pltpu.VMEM((chunk, _C), jnp.float32),
            pltpu.VMEM((chunk, _C), jnp.float32),
            pltpu.SemaphoreType.DMA,
            pltpu.SemaphoreType.DMA,
            pltpu.SemaphoreType.DMA,
            pltpu.SemaphoreType.DMA,
        ],
        compiler_params=pltpu.CompilerParams(needs_layout_passes=False),
    )
    def k(table_hbm, idx_hbm, out_hbm, idx_v, rows0, rows1, gs0, gs1, ws0, ws1):
        wid = lax.axis_index("s") * info.num_cores + lax.axis_index("c")
        g0 = wid * per_w
        batch = g0 // _T
        t0 = g0 % _T

        pltpu.sync_copy(idx_hbm.at[pl.ds(g0, per_w)], idx_v)

        rows = (rows0, rows1)
        gsem = (gs0, gs1)
        wsem = (ws0, ws1)

        def start_gather(j):
            return pltpu.async_copy(
                table_hbm.at[idx_v.at[pl.ds(j * chunk, chunk)]],
                rows[j % 2],
                gsem[j % 2],
            )

        def start_write(j):
            return pltpu.async_copy(
                rows[j % 2],
                out_hbm.at[batch, pl.ds(t0 + j * chunk, chunk), :],
                wsem[j % 2],
            )

        gd = [None] * nchunks
        wd = [None] * nchunks
        gd[0] = start_gather(0)
        for j in range(nchunks):
            if j + 1 < nchunks:
                if j - 1 >= 0:
                    wd[j - 1].wait()
                gd[j + 1] = start_gather(j + 1)
            gd[j].wait()
            wd[j] = start_write(j)
        wd[nchunks - 2].wait()
        wd[nchunks - 1].wait()

    return k(table, idx)


def kernel(tk_codes, feat_grid):
    tk = tk_codes.astype(jnp.int32)
    b_off = (jnp.arange(_B, dtype=jnp.int32) * _HW)[:, None]
    idx2d = (tk[..., 1] * _W + tk[..., 0] + b_off)[:, None, :]
    table, idx = _tc_prep(feat_grid, idx2d)
    return _sc_gather(table, idx)


# speedup vs baseline: 3.2275x; 1.0364x over previous
import functools

import jax
import jax.numpy as jnp
from jax import lax
from jax.experimental import pallas as pl
from jax.experimental.pallas import tpu as pltpu
from jax.experimental.pallas import tpu_sc as plsc

_B = 8
_C = 128
_H = 256
_W = 256
_T = 16384
_HW = _H * _W
_TOT = _B * _T

_YB = 64


def _tc_prep(feat, idx2d):
    eye = jnp.eye(_C, dtype=jnp.float32)
    blk = _YB * _W
    tblk = _T // (_H // _YB)

    def body(in_ref, idx_in_ref, eye_ref, out_ref, idx_ref):
        a = in_ref[0].reshape(_C, blk)
        out_ref[...] = lax.dot_general(
            a,
            eye_ref[...],
            (((0,), (0,)), ((), ())),
            preferred_element_type=jnp.float32,
        )
        idx_ref[...] = idx_in_ref[0, 0]

    return pl.pallas_call(
        body,
        grid=(_B, _H // _YB),
        in_specs=[
            pl.BlockSpec((1, _C, _YB, _W), lambda b, j: (b, 0, j, 0)),
            pl.BlockSpec((1, 1, tblk), lambda b, j: (b, 0, j)),
            pl.BlockSpec((_C, _C), lambda b, j: (0, 0)),
        ],
        out_specs=[
            pl.BlockSpec((blk, _C), lambda b, j: (b * (_H // _YB) + j, 0)),
            pl.BlockSpec((tblk,), lambda b, j: (b * (_H // _YB) + j,)),
        ],
        out_shape=[
            jax.ShapeDtypeStruct((_B * _HW, _C), jnp.float32),
            jax.ShapeDtypeStruct((_TOT,), jnp.int32),
        ],
    )(feat, idx2d, eye)


def _sc_gather(table, idx):
    info = plsc.get_sparse_core_info()
    nw = info.num_cores * info.num_subcores
    per_w = _TOT // nw
    chunk = 256
    nchunks = per_w // chunk

    mesh = plsc.VectorSubcoreMesh(core_axis_name="c", subcore_axis_name="s")

    @functools.partial(
        pl.kernel,
        mesh=mesh,
        out_type=jax.ShapeDtypeStruct((_B, _T, _C), jnp.float32),
        scratch_types=[
            pltpu.VMEM((per_w,), jnp.int32),
            pltpu.VMEM((chunk, _C), jnp.float32),
            pltpu.VMEM((chunk, _C), jnp.float32),
            pltpu.SemaphoreType.DMA,
            pltpu.SemaphoreType.DMA,
            pltpu.SemaphoreType.DMA,
            pltpu.SemaphoreType.DMA,
        ],
        compiler_params=pltpu.CompilerParams(needs_layout_passes=False),
    )
    def k(table_hbm, idx_hbm, out_hbm, idx_v, rows0, rows1, gs0, gs1, ws0, ws1):
        wid = lax.axis_index("s") * info.num_cores + lax.axis_index("c")
        g0 = wid * per_w
        batch = g0 // _T
        t0 = g0 % _T

        pltpu.sync_copy(idx_hbm.at[pl.ds(g0, per_w)], idx_v)

        rows = (rows0, rows1)
        gsem = (gs0, gs1)
        wsem = (ws0, ws1)

        def start_gather(j):
            return pltpu.async_copy(
                table_hbm.at[idx_v.at[pl.ds(j * chunk, chunk)]],
                rows[j % 2],
                gsem[j % 2],
            )

        def start_write(j):
            return pltpu.async_copy(
                rows[j % 2],
                out_hbm.at[batch, pl.ds(t0 + j * chunk, chunk), :],
                wsem[j % 2],
            )

        gd = [None] * nchunks
        wd = [None] * nchunks
        gd[0] = start_gather(0)
        for j in range(nchunks):
            if j + 1 < nchunks:
                if j - 1 >= 0:
                    wd[j - 1].wait()
                gd[j + 1] = start_gather(j + 1)
            gd[j].wait()
            wd[j] = start_write(j)
        wd[nchunks - 2].wait()
        wd[nchunks - 1].wait()

    return k(table, idx)


def kernel(tk_codes, feat_grid):
    tk = tk_codes.astype(jnp.int32)
    b_off = (jnp.arange(_B, dtype=jnp.int32) * _HW)[:, None]
    idx2d = (tk[..., 1] * _W + tk[..., 0] + b_off)[:, None, :]
    table, idx = _tc_prep(feat_grid, idx2d)
    return _sc_gather(table, idx)


# speedup vs baseline: 3.2326x; 1.0016x over previous
import functools

import jax
import jax.numpy as jnp
from jax import lax
from jax.experimental import pallas as pl
from jax.experimental.pallas import tpu as pltpu
from jax.experimental.pallas import tpu_sc as plsc

_B = 8
_C = 128
_H = 256
_W = 256
_T = 16384
_HW = _H * _W
_TOT = _B * _T

_YB = 64


def _tc_prep(feat, idx2d):
    eye = jnp.eye(_C, dtype=jnp.float32)
    blk = _YB * _W
    tblk = _T // (_H // _YB)

    def body(in_ref, idx_in_ref, eye_ref, out_ref, idx_ref):
        a = in_ref[0].reshape(_C, blk)
        out_ref[...] = lax.dot_general(
            a,
            eye_ref[...],
            (((0,), (0,)), ((), ())),
            preferred_element_type=jnp.float32,
        )
        idx_ref[...] = idx_in_ref[0, 0]

    return pl.pallas_call(
        body,
        grid=(_B, _H // _YB),
        in_specs=[
            pl.BlockSpec((1, _C, _YB, _W), lambda b, j: (b, 0, j, 0)),
            pl.BlockSpec((1, 1, tblk), lambda b, j: (b, 0, j)),
            pl.BlockSpec((_C, _C), lambda b, j: (0, 0)),
        ],
        out_specs=[
            pl.BlockSpec((blk, _C), lambda b, j: (b * (_H // _YB) + j, 0)),
            pl.BlockSpec((tblk,), lambda b, j: (b * (_H // _YB) + j,)),
        ],
        out_shape=[
            jax.ShapeDtypeStruct((_B * _HW, _C), jnp.float32),
            jax.ShapeDtypeStruct((_TOT,), jnp.int32),
        ],
        compiler_params=pltpu.CompilerParams(
            dimension_semantics=("parallel", "arbitrary")
        ),
    )(feat, idx2d, eye)


def _sc_gather(table, idx):
    info = plsc.get_sparse_core_info()
    nw = info.num_cores * info.num_subcores
    per_w = _TOT // nw
    chunk = 256
    nchunks = per_w // chunk

    mesh = plsc.VectorSubcoreMesh(core_axis_name="c", subcore_axis_name="s")

    @functools.partial(
        pl.kernel,
        mesh=mesh,
        out_type=jax.ShapeDtypeStruct((_B, _T, _C), jnp.float32),
        scratch_types=[
            pltpu.VMEM((per_w,), jnp.int32),
            pltpu.VMEM((chunk, _C), jnp.float32),
            pltpu.VMEM((chunk, _C), jnp.float32),
            pltpu.SemaphoreType.DMA,
            pltpu.SemaphoreType.DMA,
            pltpu.SemaphoreType.DMA,
            pltpu.SemaphoreType.DMA,
        ],
        compiler_params=pltpu.CompilerParams(needs_layout_passes=False),
    )
    def k(table_hbm, idx_hbm, out_hbm, idx_v, rows0, rows1, gs0, gs1, ws0, ws1):
        wid = lax.axis_index("s") * info.num_cores + lax.axis_index("c")
        g0 = wid * per_w
        batch = g0 // _T
        t0 = g0 % _T

        pltpu.sync_copy(idx_hbm.at[pl.ds(g0, per_w)], idx_v)

        rows = (rows0, rows1)
        gsem = (gs0, gs1)
        wsem = (ws0, ws1)

        def start_gather(j):
            return pltpu.async_copy(
                table_hbm.at[idx_v.at[pl.ds(j * chunk, chunk)]],
                rows[j % 2],
                gsem[j % 2],
            )

        def start_write(j):
            return pltpu.async_copy(
                rows[j % 2],
                out_hbm.at[batch, pl.ds(t0 + j * chunk, chunk), :],
                wsem[j % 2],
            )

        gd = [None] * nchunks
        wd = [None] * nchunks
        gd[0] = start_gather(0)
        for j in range(nchunks):
            if j + 1 < nchunks:
                if j - 1 >= 0:
                    wd[j - 1].wait()
                gd[j + 1] = start_gather(j + 1)
            gd[j].wait()
            wd[j] = start_write(j)
        wd[nchunks - 2].wait()
        wd[nchunks - 1].wait()

    return k(table, idx)


def kernel(tk_codes, feat_grid):
    tk = tk_codes.astype(jnp.int32)
    b_off = (jnp.arange(_B, dtype=jnp.int32) * _HW)[:, None]
    idx2d = (tk[..., 1] * _W + tk[..., 0] + b_off)[:, None, :]
    table, idx = _tc_prep(feat_grid, idx2d)
    return _sc_gather(table, idx)
